# baseline (device time: 8345 ns/iter reference)
import jax
import jax.numpy as jnp
from jax import lax
from jax.experimental import pallas as pl
from jax.experimental.pallas import tpu as pltpu

N_BLOCKS = 8


def kernel(x):
    m, n = x.shape
    mb = m // N_BLOCKS

    def body(x_ref, out_ref, comm_ref, send_sem, recv_sem):
        step = pl.program_id(0)
        my_x = lax.axis_index("x")
        my_y = lax.axis_index("y")
        partner = (1 - my_x, my_y)

        blk_max = jnp.max(x_ref[:, :], axis=0, keepdims=True)

        @pl.when(step == 0)
        def _():
            comm_ref[0, :, :] = blk_max

        @pl.when(step > 0)
        def _():
            comm_ref[0, :, :] = jnp.maximum(comm_ref[0, :, :], blk_max)

        @pl.when(step == N_BLOCKS - 1)
        def _():
            barrier_sem = pltpu.get_barrier_semaphore()
            pl.semaphore_signal(
                barrier_sem,
                inc=1,
                device_id=partner,
                device_id_type=pl.DeviceIdType.MESH,
            )
            pl.semaphore_wait(barrier_sem, 1)

            rdma = pltpu.make_async_remote_copy(
                src_ref=comm_ref.at[0],
                dst_ref=comm_ref.at[1],
                send_sem=send_sem,
                recv_sem=recv_sem,
                device_id=partner,
                device_id_type=pl.DeviceIdType.MESH,
            )
            rdma.start()
            rdma.wait()

            out_ref[:, :] = jnp.maximum(comm_ref[0, :, :], comm_ref[1, :, :])

    return pl.pallas_call(
        body,
        grid=(N_BLOCKS,),
        out_shape=jax.ShapeDtypeStruct((1, n), x.dtype),
        in_specs=[pl.BlockSpec((mb, n), lambda i: (i, 0))],
        out_specs=pl.BlockSpec((1, n), lambda i: (0, 0)),
        scratch_shapes=[
            pltpu.VMEM((2, 1, n), x.dtype),
            pltpu.SemaphoreType.DMA,
            pltpu.SemaphoreType.DMA,
        ],
        compiler_params=pltpu.CompilerParams(collective_id=0),
    )(x)


# device time: 7829 ns/iter; 1.0659x vs baseline; 1.0659x over previous
import jax
import jax.numpy as jnp
from jax import lax
from jax.experimental import pallas as pl
from jax.experimental.pallas import tpu as pltpu

N_CHUNKS = 4


def kernel(x):
    m, n = x.shape
    x = pltpu.with_memory_space_constraint(x, pltpu.MemorySpace.HBM)
    rows = m // N_CHUNKS

    def body(x_hbm, out_ref, buf, comm_ref, copy_sems, send_sem, recv_sem):
        my_x = lax.axis_index("x")
        my_y = lax.axis_index("y")
        partner = (1 - my_x, my_y)

        barrier_sem = pltpu.get_barrier_semaphore()
        pl.semaphore_signal(
            barrier_sem,
            inc=1,
            device_id=partner,
            device_id_type=pl.DeviceIdType.MESH,
        )

        def chunk_copy(c):
            return pltpu.make_async_copy(
                x_hbm.at[pl.ds(c * rows, rows), :],
                buf.at[c],
                copy_sems.at[c],
            )

        for c in range(N_CHUNKS):
            chunk_copy(c).start()
        for c in range(N_CHUNKS):
            chunk_copy(c).wait()
            blk_max = jnp.max(buf[c], axis=0, keepdims=True)
            if c == 0:
                comm_ref[0, :, :] = blk_max
            else:
                comm_ref[0, :, :] = jnp.maximum(comm_ref[0, :, :], blk_max)

        pl.semaphore_wait(barrier_sem, 1)
        rdma = pltpu.make_async_remote_copy(
            src_ref=comm_ref.at[0],
            dst_ref=comm_ref.at[1],
            send_sem=send_sem,
            recv_sem=recv_sem,
            device_id=partner,
            device_id_type=pl.DeviceIdType.MESH,
        )
        rdma.start()
        rdma.wait()

        out_ref[:, :] = jnp.maximum(comm_ref[0, :, :], comm_ref[1, :, :])

    return pl.pallas_call(
        body,
        out_shape=jax.ShapeDtypeStruct((1, n), x.dtype),
        in_specs=[pl.BlockSpec(memory_space=pltpu.MemorySpace.HBM)],
        out_specs=pl.BlockSpec(memory_space=pltpu.VMEM),
        scratch_shapes=[
            pltpu.VMEM((N_CHUNKS, rows, n), x.dtype),
            pltpu.VMEM((2, 1, n), x.dtype),
            pltpu.SemaphoreType.DMA((N_CHUNKS,)),
            pltpu.SemaphoreType.DMA,
            pltpu.SemaphoreType.DMA,
        ],
        compiler_params=pltpu.CompilerParams(collective_id=0),
    )(x)
